# Initial kernel scaffold; baseline (speedup 1.0000x reference)
import jax
import jax.numpy as jnp
from jax.experimental import pallas as pl


def kernel(text, embed_table, W, b):
    raise NotImplementedError("write your pallas kernel here")


# speedup vs baseline: 8.6860x; 8.6860x over previous
import functools

import jax
import jax.numpy as jnp
from jax import lax
from jax.experimental import pallas as pl
from jax.experimental.pallas import tpu as pltpu
from jax.experimental.pallas import tpu_sc as plsc


def _pool_kernel(B, S, D, NC, NS):
    NW = NC * NS
    bpw = B // NW
    mesh = plsc.VectorSubcoreMesh(core_axis_name="c", subcore_axis_name="s")

    @functools.partial(
        pl.kernel,
        out_type=jax.ShapeDtypeStruct((B, D), jnp.float32),
        mesh=mesh,
        scratch_types=[
            pltpu.VMEM((S, bpw), jnp.int32),
            pltpu.VMEM((bpw, D), jnp.float32),
            pltpu.SemaphoreType.DMA,
        ],
    )
    def pool(idx_hbm, table_hbm, out_hbm, idx_v, acc_v, sem):
        wid = lax.axis_index("s") * NC + lax.axis_index("c")
        pltpu.sync_copy(idx_hbm.at[wid], idx_v)
        pltpu.async_copy(table_hbm.at[idx_v.at[0]], acc_v, sem).wait()

        def body(l, carry):
            pltpu.async_copy(table_hbm.at[idx_v.at[l]], acc_v, sem, add=True).wait()
            return carry

        lax.fori_loop(1, S, body, 0)
        pltpu.sync_copy(acc_v, out_hbm.at[pl.ds(wid * bpw, bpw)])

    return pool


def _matmul(x, W, b2, scale, BB):
    B, D = x.shape
    C = W.shape[1]

    def mm(x_ref, w_ref, b_ref, o_ref):
        o_ref[...] = (
            jnp.dot(x_ref[...] * scale, w_ref[...], preferred_element_type=jnp.float32)
            + b_ref[...]
        )

    return pl.pallas_call(
        mm,
        grid=(B // BB,),
        in_specs=[
            pl.BlockSpec((BB, D), lambda i: (i, 0)),
            pl.BlockSpec((D, C), lambda i: (0, 0)),
            pl.BlockSpec((1, C), lambda i: (0, 0)),
        ],
        out_specs=pl.BlockSpec((BB, C), lambda i: (i, 0)),
        out_shape=jax.ShapeDtypeStruct((B, C), jnp.float32),
    )(x, W, b2)


def kernel(text, embed_table, W, b):
    B, S = text.shape
    V, D = embed_table.shape
    C = W.shape[1]
    try:
        info = plsc.get_sparse_core_info()
        NC, NS = info.num_cores, info.num_subcores
    except Exception:
        NC, NS = 2, 16
    NW = NC * NS
    bpw = B // NW
    idx = text.reshape(NW, bpw, S).transpose(0, 2, 1)
    pooled = _pool_kernel(B, S, D, NC, NS)(idx, embed_table)
    return _matmul(pooled, W, b.reshape(1, C), 1.0 / S, 512)


# speedup vs baseline: 13.9543x; 1.6065x over previous
import functools

import jax
import jax.numpy as jnp
from jax import lax
from jax.experimental import pallas as pl
from jax.experimental.pallas import tpu as pltpu
from jax.experimental.pallas import tpu_sc as plsc


def _pool_kernel(B, S, D, NC, NS):
    NW = NC * NS
    bpw = B // NW
    mesh = plsc.VectorSubcoreMesh(core_axis_name="c", subcore_axis_name="s")
    K = 8

    @functools.partial(
        pl.kernel,
        out_type=jax.ShapeDtypeStruct((B, D), jnp.float32),
        mesh=mesh,
        scratch_types=[
            pltpu.VMEM((S, bpw), jnp.int32),
            pltpu.VMEM((bpw, D), jnp.float32),
            pltpu.SemaphoreType.DMA,
        ],
    )
    def pool(idx_hbm, table_hbm, out_hbm, idx_v, acc_v, sem):
        wid = lax.axis_index("s") * NC + lax.axis_index("c")
        pltpu.sync_copy(idx_hbm.at[wid], idx_v)
        pltpu.async_copy(table_hbm.at[idx_v.at[0]], acc_v, sem).wait()

        def chunk(c, carry):
            base = 1 + c * K
            ds = [
                pltpu.async_copy(table_hbm.at[idx_v.at[base + j]], acc_v, sem, add=True)
                for j in range(K)
            ]
            for d in ds:
                d.wait()
            return carry

        nfull = (S - 1) // K
        lax.fori_loop(0, nfull, chunk, 0)
        tail = [
            pltpu.async_copy(table_hbm.at[idx_v.at[1 + nfull * K + j]], acc_v, sem, add=True)
            for j in range((S - 1) % K)
        ]
        for d in tail:
            d.wait()
        pltpu.sync_copy(acc_v, out_hbm.at[pl.ds(wid * bpw, bpw)])

    return pool


def _matmul(x, W, b2, scale, BB):
    B, D = x.shape
    C = W.shape[1]

    def mm(x_ref, w_ref, b_ref, o_ref):
        o_ref[...] = (
            jnp.dot(x_ref[...] * scale, w_ref[...], preferred_element_type=jnp.float32)
            + b_ref[...]
        )

    return pl.pallas_call(
        mm,
        grid=(B // BB,),
        in_specs=[
            pl.BlockSpec((BB, D), lambda i: (i, 0)),
            pl.BlockSpec((D, C), lambda i: (0, 0)),
            pl.BlockSpec((1, C), lambda i: (0, 0)),
        ],
        out_specs=pl.BlockSpec((BB, C), lambda i: (i, 0)),
        out_shape=jax.ShapeDtypeStruct((B, C), jnp.float32),
    )(x, W, b2)


def kernel(text, embed_table, W, b):
    B, S = text.shape
    V, D = embed_table.shape
    C = W.shape[1]
    try:
        info = plsc.get_sparse_core_info()
        NC, NS = info.num_cores, info.num_subcores
    except Exception:
        NC, NS = 2, 16
    NW = NC * NS
    bpw = B // NW
    idx = text.reshape(NW, bpw, S).transpose(0, 2, 1)
    pooled = _pool_kernel(B, S, D, NC, NS)(idx, embed_table)
    return _matmul(pooled, W, b.reshape(1, C), 1.0 / S, 512)


# speedup vs baseline: 14.8084x; 1.0612x over previous
import functools

import jax
import jax.numpy as jnp
from jax import lax
from jax.experimental import pallas as pl
from jax.experimental.pallas import tpu as pltpu
from jax.experimental.pallas import tpu_sc as plsc


def _pool_kernel(B, S, D, NC, NS):
    NW = NC * NS
    bpw = B // NW
    mesh = plsc.VectorSubcoreMesh(core_axis_name="c", subcore_axis_name="s")
    K = 8

    @functools.partial(
        pl.kernel,
        out_type=jax.ShapeDtypeStruct((B, D), jnp.float32),
        mesh=mesh,
        scratch_types=[
            pltpu.VMEM((S, bpw), jnp.int32),
            pltpu.VMEM((bpw, D), jnp.float32),
            pltpu.SemaphoreType.DMA,
        ],
    )
    def pool(idx_hbm, table_hbm, out_hbm, idx_v, acc_v, sem):
        wid = lax.axis_index("s") * NC + lax.axis_index("c")
        pltpu.sync_copy(idx_hbm.at[wid], idx_v)
        pltpu.async_copy(table_hbm.at[idx_v.at[0]], acc_v, sem).wait()

        def drain(n):
            for _ in range(n):
                pltpu.make_async_copy(
                    table_hbm.at[pl.ds(0, bpw)], acc_v, sem
                ).wait()

        nfull = (S - 1) // K
        rem = (S - 1) % K
        for j in range(K):
            pltpu.async_copy(table_hbm.at[idx_v.at[1 + j]], acc_v, sem, add=True)

        def chunk(c, carry):
            base = 1 + c * K
            for j in range(K):
                pltpu.async_copy(table_hbm.at[idx_v.at[base + j]], acc_v, sem, add=True)
            drain(K)
            return carry

        lax.fori_loop(1, nfull, chunk, 0)
        for j in range(rem):
            pltpu.async_copy(table_hbm.at[idx_v.at[1 + nfull * K + j]], acc_v, sem, add=True)
        drain(K + rem)
        pltpu.sync_copy(acc_v, out_hbm.at[pl.ds(wid * bpw, bpw)])

    return pool


def _matmul(x, W, b2, scale, BB):
    B, D = x.shape
    C = W.shape[1]

    def mm(x_ref, w_ref, b_ref, o_ref):
        o_ref[...] = (
            jnp.dot(x_ref[...] * scale, w_ref[...], preferred_element_type=jnp.float32)
            + b_ref[...]
        )

    return pl.pallas_call(
        mm,
        grid=(B // BB,),
        in_specs=[
            pl.BlockSpec((BB, D), lambda i: (i, 0)),
            pl.BlockSpec((D, C), lambda i: (0, 0)),
            pl.BlockSpec((1, C), lambda i: (0, 0)),
        ],
        out_specs=pl.BlockSpec((BB, C), lambda i: (i, 0)),
        out_shape=jax.ShapeDtypeStruct((B, C), jnp.float32),
    )(x, W, b2)


def kernel(text, embed_table, W, b):
    B, S = text.shape
    V, D = embed_table.shape
    C = W.shape[1]
    try:
        info = plsc.get_sparse_core_info()
        NC, NS = info.num_cores, info.num_subcores
    except Exception:
        NC, NS = 2, 16
    NW = NC * NS
    bpw = B // NW
    idx = text.reshape(NW, bpw, S).transpose(0, 2, 1)
    pooled = _pool_kernel(B, S, D, NC, NS)(idx, embed_table)
    return _matmul(pooled, W, b.reshape(1, C), 1.0 / S, 512)
